# baseline (speedup 1.0000x reference)
import functools

import jax
import jax.numpy as jnp
from jax import lax
from jax.experimental import pallas as pl
from jax.experimental.pallas import tpu as pltpu
from jax.experimental.pallas import tpu_sc as plsc

_VOCAB = 1000000
_EMBED_DIM = 64
_HIDDEN = 128
_OUTPUT = 2
_SEQ_LEN = 16384

_NC = 2
_NS = 16
_NW = _NC * _NS
_PER_W = _SEQ_LEN // _NW
_CH = 128
_NCHUNK = _PER_W // _CH
_L = 16


def _gather_sum_kernel(idx_hbm, table_hbm, out_hbm, idx_v, rows_v, acc_v, sem):
    c = lax.axis_index("c")
    s = lax.axis_index("s")
    wid = s * _NC + c

    pltpu.sync_copy(idx_hbm.at[wid], idx_v)

    handles = [
        pltpu.async_copy(table_hbm.at[idx_v.at[j]], rows_v.at[j], sem)
        for j in range(_NCHUNK)
    ]
    for h in handles:
        h.wait()

    accs = tuple(jnp.zeros((_L,), jnp.float32) for _ in range(_EMBED_DIM // _L))
    for j in range(_NCHUNK):
        def body(i, a, j=j):
            return tuple(
                a[k] + rows_v[j, i, pl.ds(_L * k, _L)]
                for k in range(_EMBED_DIM // _L)
            )
        accs = lax.fori_loop(0, _CH, body, accs)

    for k in range(_EMBED_DIM // _L):
        acc_v[pl.ds(_L * k, _L)] = accs[k]
    pltpu.sync_copy(acc_v, out_hbm.at[wid])


_gather_sum = functools.partial(
    pl.kernel,
    out_type=jax.ShapeDtypeStruct((_NW, _EMBED_DIM), jnp.float32),
    mesh=plsc.VectorSubcoreMesh(core_axis_name="c", subcore_axis_name="s"),
    compiler_params=pltpu.CompilerParams(use_tc_tiling_on_sc=False),
    scratch_types=[
        pltpu.VMEM((_NCHUNK, _CH), jnp.int32),
        pltpu.VMEM((_NCHUNK, _CH, _EMBED_DIM), jnp.float32),
        pltpu.VMEM((_EMBED_DIM,), jnp.float32),
        pltpu.SemaphoreType.DMA,
    ],
)(_gather_sum_kernel)


def _mlp_kernel(ps_ref, vwt_ref, vb_ref, wwt_ref, wb_ref, o_ref):
    avg = jnp.sum(ps_ref[...], axis=0, keepdims=True) * (1.0 / _SEQ_LEN)
    h = jnp.tanh(
        jnp.dot(avg, vwt_ref[...], precision=lax.Precision.HIGHEST)
        + vb_ref[...]
    )
    o = jnp.dot(h, wwt_ref[...], precision=lax.Precision.HIGHEST) + wb_ref[...]
    m = jnp.max(o, axis=1, keepdims=True)
    e = o - m
    lse = jnp.log(jnp.sum(jnp.exp(e), axis=1, keepdims=True))
    o_ref[...] = e - lse


def kernel(x, table, V_w, V_b, W_w, W_b):
    idx = x.astype(jnp.int32).reshape(_NW, _NCHUNK, _CH)
    psums = _gather_sum(idx, table)
    out = pl.pallas_call(
        _mlp_kernel,
        out_shape=jax.ShapeDtypeStruct((1, _OUTPUT), jnp.float32),
    )(
        psums,
        V_w.T,
        V_b.reshape(1, _HIDDEN),
        W_w.T,
        W_b.reshape(1, _OUTPUT),
    )
    return out.reshape(_OUTPUT)
